# Initial kernel scaffold; baseline (speedup 1.0000x reference)
import jax
import jax.numpy as jnp
from jax.experimental import pallas as pl


def kernel(x, Q_latent, freqs_cis, mask, Wq_up_w, Wq_up_b, Wk_w, Wk_b, ln_g, ln_b):
    raise NotImplementedError("write your pallas kernel here")


# speedup vs baseline: 9.0472x; 9.0472x over previous
import functools

import numpy as np
import jax
import jax.numpy as jnp
from jax.experimental import pallas as pl
from jax.experimental.pallas import tpu as pltpu

B, S, DIM = 2, 2048, 2048
Q_LORA = 1536
H, D = 16, 128
R = 64
HALF = R // 2

_SCALE = float(jnp.bfloat16(D ** -0.5))


def _hadamard_matrix(d: int) -> np.ndarray:
    x = np.eye(d, dtype=np.float64)
    h = 1
    while h < d:
        x = x.reshape(-1, d // (2 * h), 2, h)
        a = x[..., 0, :]
        b = x[..., 1, :]
        x = np.stack([a + b, a - b], axis=-2)
        h *= 2
    return x.reshape(d, d)


_HAD = jnp.asarray(_hadamard_matrix(D), dtype=jnp.bfloat16)


def _rope_had(v, cos, sa, sb, had):
    v = v * cos + pltpu.roll(v, D - HALF, axis=1) * sa + pltpu.roll(v, HALF, axis=1) * sb
    vb = v.astype(jnp.bfloat16)
    return (jnp.dot(vb, had, preferred_element_type=jnp.float32)
            * _SCALE).astype(jnp.bfloat16)


def _body(ql_ref, x_ref, wq_ref, wk_ref, bq_ref, bk_ref, g_ref, beta_ref,
          cos_ref, sa_ref, sb_ref, had_ref, qo_ref, ko_ref):
    cos = cos_ref[...]
    sa = sa_ref[...]
    sb = sb_ref[...]
    had = had_ref[...]

    k = jnp.dot(x_ref[0], wk_ref[...], preferred_element_type=jnp.float32)
    k = k + bk_ref[...]
    mu = jnp.mean(k, axis=-1, keepdims=True)
    var = jnp.mean((k - mu) ** 2, axis=-1, keepdims=True)
    k = (k - mu) / jnp.sqrt(var + 1e-5) * g_ref[...] + beta_ref[...]
    ko_ref[0] = _rope_had(k, cos, sa, sb, had)

    q2 = jnp.dot(ql_ref[0], wq_ref[...], preferred_element_type=jnp.float32)
    q2 = q2 + bq_ref[...]
    for h in range(H):
        sl = slice(h * D, (h + 1) * D)
        qo_ref[0, :, sl] = _rope_had(q2[:, sl], cos, sa, sb, had)


@functools.partial(jax.jit, static_argnames=())
def kernel(x, Q_latent, freqs_cis, mask, Wq_up_w, Wq_up_b, Wk_w, Wk_b, ln_g, ln_b):
    del mask
    T = 256

    cos32 = freqs_cis[..., 0]
    sin32 = freqs_cis[..., 1]
    ones = jnp.ones((S, D - R), dtype=jnp.float32)
    zeros = jnp.zeros((S, D - R), dtype=jnp.float32)
    z32 = jnp.zeros((S, HALF), dtype=jnp.float32)
    cos_row = jnp.concatenate([cos32, cos32, ones], axis=1)
    sa_row = jnp.concatenate([-sin32, z32, zeros], axis=1)
    sb_row = jnp.concatenate([z32, sin32, zeros], axis=1)

    wqT = Wq_up_w.T
    wkT = Wk_w.T
    bq = Wq_up_b.reshape(1, H * D)
    bk = Wk_b.reshape(1, D)
    g = ln_g.reshape(1, D)
    beta = ln_b.reshape(1, D)

    grid = (B, S // T)
    qout, kout = pl.pallas_call(
        _body,
        grid=grid,
        in_specs=[
            pl.BlockSpec((1, T, Q_LORA), lambda b, i: (b, i, 0)),
            pl.BlockSpec((1, T, DIM), lambda b, i: (b, i, 0)),
            pl.BlockSpec((Q_LORA, H * D), lambda b, i: (0, 0)),
            pl.BlockSpec((DIM, D), lambda b, i: (0, 0)),
            pl.BlockSpec((1, H * D), lambda b, i: (0, 0)),
            pl.BlockSpec((1, D), lambda b, i: (0, 0)),
            pl.BlockSpec((1, D), lambda b, i: (0, 0)),
            pl.BlockSpec((1, D), lambda b, i: (0, 0)),
            pl.BlockSpec((T, D), lambda b, i: (i, 0)),
            pl.BlockSpec((T, D), lambda b, i: (i, 0)),
            pl.BlockSpec((T, D), lambda b, i: (i, 0)),
            pl.BlockSpec((D, D), lambda b, i: (0, 0)),
        ],
        out_specs=[
            pl.BlockSpec((1, T, H * D), lambda b, i: (b, i, 0)),
            pl.BlockSpec((1, T, D), lambda b, i: (b, i, 0)),
        ],
        out_shape=[
            jax.ShapeDtypeStruct((B, S, H * D), jnp.bfloat16),
            jax.ShapeDtypeStruct((B, S, D), jnp.bfloat16),
        ],
        compiler_params=pltpu.CompilerParams(
            dimension_semantics=("parallel", "parallel"),
        ),
    )(Q_latent, x, wqT, wkT, bq, bk, g, beta, cos_row, sa_row, sb_row, _HAD)

    return qout.reshape(B, S, H, D), kout


# speedup vs baseline: 9.7963x; 1.0828x over previous
import functools

import numpy as np
import jax
import jax.numpy as jnp
from jax.experimental import pallas as pl
from jax.experimental.pallas import tpu as pltpu

B, S, DIM = 2, 2048, 2048
Q_LORA = 1536
H, D = 16, 128
R = 64
HALF = R // 2

_SCALE = float(jnp.bfloat16(D ** -0.5))


def _hadamard_matrix(d: int) -> np.ndarray:
    x = np.eye(d, dtype=np.float64)
    h = 1
    while h < d:
        x = x.reshape(-1, d // (2 * h), 2, h)
        a = x[..., 0, :]
        b = x[..., 1, :]
        x = np.stack([a + b, a - b], axis=-2)
        h *= 2
    return x.reshape(d, d)


_HAD = jnp.asarray(_hadamard_matrix(D), dtype=jnp.bfloat16)


def _rope_had(v, cos, sa, sb, had):
    v = v * cos + pltpu.roll(v, D - HALF, axis=1) * sa + pltpu.roll(v, HALF, axis=1) * sb
    vb = v.astype(jnp.bfloat16)
    return (jnp.dot(vb, had, preferred_element_type=jnp.float32)
            * _SCALE).astype(jnp.bfloat16)


def _body(ql_ref, x_ref, wq_ref, wk_ref, bq_ref, bk_ref, g_ref, beta_ref,
          cos_ref, sa_ref, sb_ref, had_ref, qo_ref, ko_ref):
    cos = cos_ref[...]
    sa = sa_ref[...]
    sb = sb_ref[...]
    had = had_ref[...]

    k = jnp.dot(x_ref[0].astype(jnp.bfloat16), wk_ref[...],
                preferred_element_type=jnp.float32)
    k = k + bk_ref[...]
    mu = jnp.mean(k, axis=-1, keepdims=True)
    var = jnp.mean((k - mu) ** 2, axis=-1, keepdims=True)
    k = (k - mu) / jnp.sqrt(var + 1e-5) * g_ref[...] + beta_ref[...]
    ko_ref[0] = _rope_had(k, cos, sa, sb, had)

    q2 = jnp.dot(ql_ref[0].astype(jnp.bfloat16), wq_ref[...],
                 preferred_element_type=jnp.float32)
    q2 = q2 + bq_ref[...]
    for h in range(H):
        sl = slice(h * D, (h + 1) * D)
        qo_ref[0, :, sl] = _rope_had(q2[:, sl], cos, sa, sb, had)


@functools.partial(jax.jit, static_argnames=())
def kernel(x, Q_latent, freqs_cis, mask, Wq_up_w, Wq_up_b, Wk_w, Wk_b, ln_g, ln_b):
    del mask
    T = 256

    cos32 = freqs_cis[..., 0]
    sin32 = freqs_cis[..., 1]
    ones = jnp.ones((S, D - R), dtype=jnp.float32)
    zeros = jnp.zeros((S, D - R), dtype=jnp.float32)
    z32 = jnp.zeros((S, HALF), dtype=jnp.float32)
    cos_row = jnp.concatenate([cos32, cos32, ones], axis=1)
    sa_row = jnp.concatenate([-sin32, z32, zeros], axis=1)
    sb_row = jnp.concatenate([z32, sin32, zeros], axis=1)

    wqT = Wq_up_w.T.astype(jnp.bfloat16)
    wkT = Wk_w.T.astype(jnp.bfloat16)
    bq = Wq_up_b.reshape(1, H * D)
    bk = Wk_b.reshape(1, D)
    g = ln_g.reshape(1, D)
    beta = ln_b.reshape(1, D)

    grid = (B, S // T)
    qout, kout = pl.pallas_call(
        _body,
        grid=grid,
        in_specs=[
            pl.BlockSpec((1, T, Q_LORA), lambda b, i: (b, i, 0)),
            pl.BlockSpec((1, T, DIM), lambda b, i: (b, i, 0)),
            pl.BlockSpec((Q_LORA, H * D), lambda b, i: (0, 0)),
            pl.BlockSpec((DIM, D), lambda b, i: (0, 0)),
            pl.BlockSpec((1, H * D), lambda b, i: (0, 0)),
            pl.BlockSpec((1, D), lambda b, i: (0, 0)),
            pl.BlockSpec((1, D), lambda b, i: (0, 0)),
            pl.BlockSpec((1, D), lambda b, i: (0, 0)),
            pl.BlockSpec((T, D), lambda b, i: (i, 0)),
            pl.BlockSpec((T, D), lambda b, i: (i, 0)),
            pl.BlockSpec((T, D), lambda b, i: (i, 0)),
            pl.BlockSpec((D, D), lambda b, i: (0, 0)),
        ],
        out_specs=[
            pl.BlockSpec((1, T, H * D), lambda b, i: (b, i, 0)),
            pl.BlockSpec((1, T, D), lambda b, i: (b, i, 0)),
        ],
        out_shape=[
            jax.ShapeDtypeStruct((B, S, H * D), jnp.bfloat16),
            jax.ShapeDtypeStruct((B, S, D), jnp.bfloat16),
        ],
        compiler_params=pltpu.CompilerParams(
            dimension_semantics=("parallel", "parallel"),
        ),
    )(Q_latent, x, wqT, wkT, bq, bk, g, beta, cos_row, sa_row, sb_row, _HAD)

    return qout.reshape(B, S, H, D), kout
